# baseline (speedup 1.0000x reference)
import functools

import jax
import jax.numpy as jnp
from jax import lax
from jax.experimental import pallas as pl
from jax.experimental.pallas import tpu as pltpu
from jax.experimental.pallas import tpu_sc as plsc

B, S, D = 2, 2048, 768
E = 16
FF = 3072
T = B * S
C = int(1.25 * T / E)
TRASH = E * C
BUF_ROWS = E * C + 8
G = 512
NCH = T // G
NW = 32
TPW = T // NW
NCHK = 4
CHK = TPW // NCHK
F_BLK = 3072
FB = FF // F_BLK
AUX_COEF = 0.01


def _router_body(x_ref, wr_ref, sel_ref, dst_ref, src_ref, scale_ref, loss_ref):
    x = x_ref[...].reshape(T, D)
    wr = wr_ref[...]
    logits = jnp.dot(x, wr, preferred_element_type=jnp.float32)
    lanes = lax.broadcasted_iota(jnp.int32, (T, E), 1)
    ones8 = jnp.ones((E, 8), jnp.float32)
    m = jnp.max(logits, axis=-1, keepdims=True)
    p = jnp.exp(logits - m)
    denom = jnp.dot(p, ones8, preferred_element_type=jnp.float32)[:, 0:1]
    probs = p / denom
    pmax = jnp.max(probs, axis=-1, keepdims=True)
    seli = jnp.min(jnp.where(probs == pmax, lanes, E), axis=-1, keepdims=True)
    one_hot = (lanes == seli).astype(jnp.float32)

    tri = (lax.broadcasted_iota(jnp.int32, (G, G), 0)
           >= lax.broadcasted_iota(jnp.int32, (G, G), 1)).astype(jnp.float32)
    cnt_chunks = []
    run = jnp.zeros((1, E), jnp.float32)
    for c in range(NCH):
        ohc = one_hot[c * G:(c + 1) * G]
        incl = jnp.dot(tri, ohc, preferred_element_type=jnp.float32) + run
        cnt_chunks.append(incl)
        run = incl[G - 1:G, :]
    cnt = jnp.concatenate(cnt_chunks, axis=0)
    pos = jnp.dot(cnt * one_hot, ones8,
                  preferred_element_type=jnp.float32)[:, 0:1] - 1.0

    keep = pos < C
    pos_c = jnp.minimum(pos, C - 1).astype(jnp.int32)
    src = seli * C + pos_c
    sel_ref[...] = seli.reshape(NW, TPW)
    src_ref[...] = src.reshape(NW, TPW)
    dst_ref[...] = jnp.where(keep, src, TRASH).reshape(NW, TPW)
    scale_ref[...] = jnp.where(keep, pmax, 0.0).reshape(NW, TPW)

    probs_sum = jnp.sum(probs, axis=0, keepdims=True)
    aux = jnp.sum(probs_sum * run) * (E * AUX_COEF / (T * T))
    loss_ref[0, 0] = aux


def _router(hidden, wr):
    return pl.pallas_call(
        _router_body,
        out_shape=[
            jax.ShapeDtypeStruct((NW, TPW), jnp.int32),
            jax.ShapeDtypeStruct((NW, TPW), jnp.int32),
            jax.ShapeDtypeStruct((NW, TPW), jnp.int32),
            jax.ShapeDtypeStruct((NW, TPW), jnp.float32),
            jax.ShapeDtypeStruct((1, 1), jnp.float32),
        ],
        out_specs=[
            pl.BlockSpec(memory_space=pltpu.VMEM),
            pl.BlockSpec(memory_space=pltpu.VMEM),
            pl.BlockSpec(memory_space=pltpu.VMEM),
            pl.BlockSpec(memory_space=pltpu.VMEM),
            pl.BlockSpec(memory_space=pltpu.SMEM),
        ],
    )(hidden, wr)


def _ffn_body(buf_ref, w1_ref, b1_ref, w2_ref, b2_ref, o_ref):
    f = pl.program_id(1)
    xb = buf_ref[...]
    h = jnp.dot(xb, w1_ref[0], preferred_element_type=jnp.float32) + b1_ref[0]
    h = jax.nn.gelu(h)
    acc = jnp.dot(h, w2_ref[0], preferred_element_type=jnp.float32)

    @pl.when(f == 0)
    def _():
        o_ref[...] = acc + b2_ref[0]

    @pl.when(f != 0)
    def _():
        o_ref[...] += acc


def _ffn(buf, W1, b1, W2, b2):
    return pl.pallas_call(
        _ffn_body,
        grid=(E, FB),
        in_specs=[
            pl.BlockSpec((C, D), lambda e, f: (e, 0)),
            pl.BlockSpec((1, D, F_BLK), lambda e, f: (e, 0, f)),
            pl.BlockSpec((1, 1, F_BLK), lambda e, f: (e, 0, f)),
            pl.BlockSpec((1, F_BLK, D), lambda e, f: (e, f, 0)),
            pl.BlockSpec((1, 1, D), lambda e, f: (e, 0, 0)),
        ],
        out_specs=pl.BlockSpec((C, D), lambda e, f: (e, 0)),
        out_shape=jax.ShapeDtypeStruct((E * C, D), jnp.float32),
    )(buf, W1, b1.reshape(E, 1, FF), W2, b2.reshape(E, 1, D))


@functools.cache
def _get_dispatch():
    mesh = plsc.VectorSubcoreMesh(core_axis_name="c", subcore_axis_name="s")

    @functools.partial(
        pl.kernel,
        out_type=jax.ShapeDtypeStruct((BUF_ROWS, D), jnp.float32),
        mesh=mesh,
        scratch_types=[
            pltpu.VMEM((NCHK, CHK), jnp.int32),
            pltpu.VMEM((TPW, D), jnp.float32),
            [pltpu.SemaphoreType.DMA] * NCHK,
            [pltpu.SemaphoreType.DMA] * NCHK,
        ],
    )
    def _dispatch(x_hbm, dst_hbm, buf_hbm, idx_v, rows_v, sin, sout):
        wid = lax.axis_index("s") * 2 + lax.axis_index("c")
        base = wid * TPW
        for g in range(NCHK):
            pltpu.sync_copy(dst_hbm.at[wid, pl.ds(g * CHK, CHK)], idx_v.at[g])
        hin = [
            pltpu.async_copy(
                x_hbm.at[pl.ds(base + g * CHK, CHK)],
                rows_v.at[pl.ds(g * CHK, CHK)], sin[g])
            for g in range(NCHK)
        ]
        hout = []
        for g in range(NCHK):
            hin[g].wait()
            hout.append(pltpu.async_copy(
                rows_v.at[pl.ds(g * CHK, CHK)],
                buf_hbm.at[idx_v.at[g]], sout[g]))
        for g in range(NCHK):
            hout[g].wait()

    return _dispatch


@functools.cache
def _get_combine():
    mesh = plsc.VectorSubcoreMesh(core_axis_name="c", subcore_axis_name="s")

    @functools.partial(
        pl.kernel,
        out_type=jax.ShapeDtypeStruct((T, D), jnp.float32),
        mesh=mesh,
        scratch_types=[
            pltpu.VMEM((NCHK, CHK), jnp.int32),
            pltpu.VMEM((TPW,), jnp.float32),
            pltpu.VMEM((TPW, D), jnp.float32),
            [pltpu.SemaphoreType.DMA] * NCHK,
            [pltpu.SemaphoreType.DMA] * NCHK,
        ],
    )
    def _combine(eo_hbm, src_hbm, scale_hbm, y_hbm, idx_v, scale_v, rows_v,
                 sin, sout):
        wid = lax.axis_index("s") * 2 + lax.axis_index("c")
        base = wid * TPW
        for g in range(NCHK):
            pltpu.sync_copy(src_hbm.at[wid, pl.ds(g * CHK, CHK)], idx_v.at[g])
        pltpu.sync_copy(scale_hbm.at[wid], scale_v)
        hin = [
            pltpu.async_copy(
                eo_hbm.at[idx_v.at[g]],
                rows_v.at[pl.ds(g * CHK, CHK)], sin[g])
            for g in range(NCHK)
        ]
        hout = []
        for g in range(NCHK):
            hin[g].wait()

            @plsc.parallel_loop(g * CHK, (g + 1) * CHK, step=1, unroll=2)
            def _(r):
                grp = (r // 16) * 16
                s16 = scale_v[pl.ds(grp, 16)]
                s = s16[jnp.full((16,), r - grp, jnp.int32)]
                for j in range(D // 16):
                    rows_v[r, pl.ds(j * 16, 16)] = (
                        rows_v[r, pl.ds(j * 16, 16)] * s)

            hout.append(pltpu.async_copy(
                rows_v.at[pl.ds(g * CHK, CHK)],
                y_hbm.at[pl.ds(base + g * CHK, CHK)], sout[g]))
        for g in range(NCHK):
            hout[g].wait()

    return _combine


def kernel(hidden_states, Wr, W1, b1, W2, b2):
    x = hidden_states.reshape(T, D)
    sel2, dst2, src2, scale2, loss = _router(hidden_states, Wr)
    buf = _get_dispatch()(x, dst2)
    eo = _ffn(buf, W1, b1, W2, b2)
    y = _get_combine()(eo, src2, scale2)
    output = y.reshape(B, S, D)
    routing_info = sel2.reshape(B, S)
    total_loss = loss[0, 0]
    return (output, routing_info, total_loss)


# speedup vs baseline: 1.0563x; 1.0563x over previous
import functools

import jax
import jax.numpy as jnp
from jax import lax
from jax.experimental import pallas as pl
from jax.experimental.pallas import tpu as pltpu
from jax.experimental.pallas import tpu_sc as plsc

B, S, D = 2, 2048, 768
E = 16
FF = 3072
T = B * S
C = int(1.25 * T / E)
TRASH = E * C
BUF_ROWS = E * C + 8
G = 512
NCH = T // G
NW = 32
TPW = T // NW
NCHK = 2
CHK = TPW // NCHK
F_BLK = 3072
FB = FF // F_BLK
AUX_COEF = 0.01


def _router_body(x_ref, wr_ref, sel_ref, dst_ref, src_ref, scale_ref, loss_ref):
    x = x_ref[...].reshape(T, D)
    wr = wr_ref[...]
    logits = jnp.dot(x, wr, preferred_element_type=jnp.float32)
    lanes = lax.broadcasted_iota(jnp.int32, (T, E), 1)
    m = jnp.max(logits, axis=-1, keepdims=True)
    p = jnp.exp(logits - m)
    denom = jnp.sum(p, axis=-1, keepdims=True)
    probs = p / denom
    pmax = jnp.max(probs, axis=-1, keepdims=True)
    seli = jnp.min(jnp.where(probs == pmax, lanes, E), axis=-1, keepdims=True)
    one_hot = (lanes == seli).astype(jnp.float32)

    tri = (lax.broadcasted_iota(jnp.int32, (G, G), 0)
           >= lax.broadcasted_iota(jnp.int32, (G, G), 1)).astype(jnp.float32)
    cnt_chunks = []
    run = jnp.zeros((1, E), jnp.float32)
    for c in range(NCH):
        ohc = one_hot[c * G:(c + 1) * G]
        incl = jnp.dot(tri, ohc, preferred_element_type=jnp.float32) + run
        cnt_chunks.append(incl)
        run = incl[G - 1:G, :]
    cnt = jnp.concatenate(cnt_chunks, axis=0)
    pos = jnp.sum(cnt * one_hot, axis=-1, keepdims=True) - 1.0

    keep = pos < C
    pos_c = jnp.minimum(pos, C - 1).astype(jnp.int32)
    src = seli * C + pos_c
    sel_ref[...] = seli.reshape(NW, TPW)
    src_ref[...] = src.reshape(NW, TPW)
    dst_ref[...] = jnp.where(keep, src, TRASH).reshape(NW, TPW)
    scale_ref[...] = jnp.where(keep, pmax, 0.0).reshape(NW, TPW)

    probs_sum = jnp.sum(probs, axis=0, keepdims=True)
    aux = jnp.sum(probs_sum * run) * (E * AUX_COEF / (T * T))
    loss_ref[0, 0] = aux


def _router(hidden, wr):
    return pl.pallas_call(
        _router_body,
        out_shape=[
            jax.ShapeDtypeStruct((NW, TPW), jnp.int32),
            jax.ShapeDtypeStruct((NW, TPW), jnp.int32),
            jax.ShapeDtypeStruct((NW, TPW), jnp.int32),
            jax.ShapeDtypeStruct((NW, TPW), jnp.float32),
            jax.ShapeDtypeStruct((1, 1), jnp.float32),
        ],
        out_specs=[
            pl.BlockSpec(memory_space=pltpu.VMEM),
            pl.BlockSpec(memory_space=pltpu.VMEM),
            pl.BlockSpec(memory_space=pltpu.VMEM),
            pl.BlockSpec(memory_space=pltpu.VMEM),
            pl.BlockSpec(memory_space=pltpu.SMEM),
        ],
    )(hidden, wr)


def _ffn_body(buf_ref, w1_ref, b1_ref, w2_ref, b2_ref, o_ref):
    f = pl.program_id(1)
    xb = buf_ref[...]
    h = jnp.dot(xb, w1_ref[0], preferred_element_type=jnp.float32) + b1_ref[0]
    h = jax.nn.gelu(h)
    acc = jnp.dot(h, w2_ref[0], preferred_element_type=jnp.float32)

    @pl.when(f == 0)
    def _():
        o_ref[...] = acc + b2_ref[0]

    @pl.when(f != 0)
    def _():
        o_ref[...] += acc


def _ffn(buf, W1, b1, W2, b2):
    return pl.pallas_call(
        _ffn_body,
        grid=(E, FB),
        in_specs=[
            pl.BlockSpec((C, D), lambda e, f: (e, 0)),
            pl.BlockSpec((1, D, F_BLK), lambda e, f: (e, 0, f)),
            pl.BlockSpec((1, 1, F_BLK), lambda e, f: (e, 0, f)),
            pl.BlockSpec((1, F_BLK, D), lambda e, f: (e, f, 0)),
            pl.BlockSpec((1, 1, D), lambda e, f: (e, 0, 0)),
        ],
        out_specs=pl.BlockSpec((C, D), lambda e, f: (e, 0)),
        out_shape=jax.ShapeDtypeStruct((E * C, D), jnp.float32),
    )(buf, W1, b1.reshape(E, 1, FF), W2, b2.reshape(E, 1, D))


@functools.cache
def _get_dispatch():
    mesh = plsc.VectorSubcoreMesh(core_axis_name="c", subcore_axis_name="s")

    @functools.partial(
        pl.kernel,
        out_type=jax.ShapeDtypeStruct((BUF_ROWS, D), jnp.float32),
        mesh=mesh,
        scratch_types=[
            pltpu.VMEM((NCHK, CHK), jnp.int32),
            pltpu.VMEM((TPW, D), jnp.float32),
            [pltpu.SemaphoreType.DMA] * NCHK,
            [pltpu.SemaphoreType.DMA] * NCHK,
        ],
    )
    def _dispatch(x_hbm, dst_hbm, buf_hbm, idx_v, rows_v, sin, sout):
        wid = lax.axis_index("s") * 2 + lax.axis_index("c")
        base = wid * TPW
        for g in range(NCHK):
            pltpu.sync_copy(dst_hbm.at[wid, pl.ds(g * CHK, CHK)], idx_v.at[g])
        hin = [
            pltpu.async_copy(
                x_hbm.at[pl.ds(base + g * CHK, CHK)],
                rows_v.at[pl.ds(g * CHK, CHK)], sin[g])
            for g in range(NCHK)
        ]
        hout = []
        for g in range(NCHK):
            hin[g].wait()
            hout.append(pltpu.async_copy(
                rows_v.at[pl.ds(g * CHK, CHK)],
                buf_hbm.at[idx_v.at[g]], sout[g]))
        for g in range(NCHK):
            hout[g].wait()

    return _dispatch


@functools.cache
def _get_combine():
    mesh = plsc.VectorSubcoreMesh(core_axis_name="c", subcore_axis_name="s")

    @functools.partial(
        pl.kernel,
        out_type=jax.ShapeDtypeStruct((T, D), jnp.float32),
        mesh=mesh,
        scratch_types=[
            pltpu.VMEM((NCHK, CHK), jnp.int32),
            pltpu.VMEM((TPW,), jnp.float32),
            pltpu.VMEM((TPW, D), jnp.float32),
            [pltpu.SemaphoreType.DMA] * NCHK,
            [pltpu.SemaphoreType.DMA] * NCHK,
        ],
    )
    def _combine(eo_hbm, src_hbm, scale_hbm, y_hbm, idx_v, scale_v, rows_v,
                 sin, sout):
        wid = lax.axis_index("s") * 2 + lax.axis_index("c")
        base = wid * TPW
        for g in range(NCHK):
            pltpu.sync_copy(src_hbm.at[wid, pl.ds(g * CHK, CHK)], idx_v.at[g])
        pltpu.sync_copy(scale_hbm.at[wid], scale_v)
        hin = [
            pltpu.async_copy(
                eo_hbm.at[idx_v.at[g]],
                rows_v.at[pl.ds(g * CHK, CHK)], sin[g])
            for g in range(NCHK)
        ]
        hout = []
        for g in range(NCHK):
            hin[g].wait()

            @plsc.parallel_loop(g * CHK, (g + 1) * CHK, step=1, unroll=2)
            def _(r):
                grp = (r // 16) * 16
                s16 = scale_v[pl.ds(grp, 16)]
                s = s16[jnp.full((16,), r - grp, jnp.int32)]
                for j in range(D // 16):
                    rows_v[r, pl.ds(j * 16, 16)] = (
                        rows_v[r, pl.ds(j * 16, 16)] * s)

            hout.append(pltpu.async_copy(
                rows_v.at[pl.ds(g * CHK, CHK)],
                y_hbm.at[pl.ds(base + g * CHK, CHK)], sout[g]))
        for g in range(NCHK):
            hout[g].wait()

    return _combine


def kernel(hidden_states, Wr, W1, b1, W2, b2):
    x = hidden_states.reshape(T, D)
    sel2, dst2, src2, scale2, loss = _router(hidden_states, Wr)
    buf = _get_dispatch()(x, dst2)
    eo = _ffn(buf, W1, b1, W2, b2)
    y = _get_combine()(eo, src2, scale2)
    output = y.reshape(B, S, D)
    routing_info = sel2.reshape(B, S)
    total_loss = loss[0, 0]
    return (output, routing_info, total_loss)
